# baseline (speedup 1.0000x reference)
import jax
import jax.numpy as jnp
from jax.experimental import pallas as pl
from jax.experimental.pallas import tpu as pltpu

_IN_LEN = 64
_C1, _C2, _NOUT = 16, 32, 2
_L1 = _IN_LEN - 2
_P1 = _L1 // 2
_L2 = _P1 - 2
_P2 = _L2 // 2
_PT = 32
_M1W = _PT * _C1
_G = 8
_NG = 4
_W2K = (_G + 2) * _C1
_W2N = 2 * _G * _C1
_FCK = _NG * 4 * _C2
_NPAD = 128
_OUTW = 8


def _pack(w1, b1, w2, b2, fcw, fcb):
    f32 = jnp.float32
    w1 = w1.astype(f32)

    r = jnp.arange(_IN_LEN)[:, None]
    col = jnp.arange(_M1W)[None, :]
    tp = col // _C1
    valid = tp <= _P1 - 1
    w1a = jnp.zeros((_IN_LEN, _M1W), f32)
    w1b = jnp.zeros((_IN_LEN, _M1W), f32)
    for k in range(3):
        vk = jnp.tile(w1[:, 0, k], _PT)[None, :]
        w1a = w1a + jnp.where((r == 2 * tp + k) & valid, vk, 0.0)
        w1b = w1b + jnp.where((r == 2 * tp + 1 + k) & valid, vk, 0.0)
    w1p = jnp.concatenate([w1a, w1b], axis=1)
    b1row = (jnp.tile(b1.astype(f32), _PT) *
             (jnp.arange(_M1W) < _P1 * _C1))[None, :]

    dtp = (jnp.arange(_W2K) // _C1)[:, None]
    cv = jnp.arange(_W2N)[None, :]
    t_loc = 2 * ((cv % 128) // _C2) + (cv >= 128)
    w2p = jnp.zeros((_W2K, _W2N), f32)
    for k in range(3):
        tile_k = jnp.tile(w2[:, :, k].astype(f32).T, (_W2K // _C1, _W2N // _C2))
        w2p = w2p + jnp.where(dtp == t_loc + k, tile_k, 0.0)
    b2row = jnp.tile(b2.astype(f32), 4)[None, :]

    f3 = jnp.transpose(fcw.astype(f32).reshape(_NOUT, _C2, _P2), (2, 1, 0))
    f3 = jnp.pad(f3, ((0, 2), (0, 0), (0, 0))).reshape(_FCK, _NOUT)
    wfc = jnp.pad(f3, ((0, 0), (0, _NPAD - _NOUT)))
    fcbrow = jnp.pad(fcb.astype(f32), (0, _NPAD - _NOUT))[None, :]

    bf16 = jnp.bfloat16
    return (w1p.astype(bf16), b1row.astype(bf16), w2p.astype(bf16),
            b2row.astype(bf16), wfc.astype(bf16), fcbrow)


def _body(x_ref, w1_ref, b1_ref, w2_ref, b2_ref, wfc_ref, fcb_ref, out_ref):
    f32 = jnp.float32
    bf16 = jnp.bfloat16
    x = x_ref[...].astype(bf16)
    y1 = jnp.dot(x, w1_ref[...], preferred_element_type=f32)
    y1 = y1.astype(bf16)
    m1 = jnp.maximum(y1[:, :_M1W], y1[:, _M1W:]) + b1_ref[...]
    m1 = jnp.maximum(m1, 0.0)
    m1 = jnp.concatenate(
        [m1, jnp.zeros((m1.shape[0], 2 * _C1 * _NG), bf16)], axis=1)

    w2 = w2_ref[...]
    parts = []
    for g in range(_NG):
        lo = 2 * _C1 * _G // 2 * g
        a2g = jnp.dot(m1[:, lo:lo + _W2K], w2,
                      preferred_element_type=f32)
        a2g = a2g.astype(bf16)
        m2g = jnp.maximum(a2g[:, :128], a2g[:, 128:]) + b2_ref[...]
        parts.append(jnp.maximum(m2g, 0.0))
    m2 = jnp.concatenate(parts, axis=1)

    y = jnp.dot(m2, wfc_ref[...], preferred_element_type=f32)
    out_ref[...] = (y + fcb_ref[...])[:, :_OUTW]


def kernel(x, w1, b1, w2, b2, fcw, fcb, tb=4096):
    B, L = x.shape
    assert L == _IN_LEN
    w1p, b1row, w2p, b2row, wfc, fcbrow = _pack(w1, b1, w2, b2, fcw, fcb)

    nblk = pl.cdiv(B, tb)
    b_pad = nblk * tb
    x_pad = jnp.pad(x.astype(jnp.float32), ((0, b_pad - B), (0, 0)))
    out = pl.pallas_call(
        _body,
        out_shape=jax.ShapeDtypeStruct((b_pad, _OUTW), jnp.float32),
        grid=(nblk,),
        in_specs=[
            pl.BlockSpec((tb, _IN_LEN), lambda i: (i, 0)),
            pl.BlockSpec((_IN_LEN, 2 * _M1W), lambda i: (0, 0)),
            pl.BlockSpec((1, _M1W), lambda i: (0, 0)),
            pl.BlockSpec((_W2K, _W2N), lambda i: (0, 0)),
            pl.BlockSpec((1, 128), lambda i: (0, 0)),
            pl.BlockSpec((_FCK, _NPAD), lambda i: (0, 0)),
            pl.BlockSpec((1, _NPAD), lambda i: (0, 0)),
        ],
        out_specs=pl.BlockSpec((tb, _OUTW), lambda i: (i, 0)),
        compiler_params=pltpu.CompilerParams(
            dimension_semantics=("parallel",),
            vmem_limit_bytes=64 * 1024 * 1024),
    )(x_pad, w1p, b1row, w2p, b2row, wfc, fcbrow)
    return out[:B, :_NOUT]


# speedup vs baseline: 1.0131x; 1.0131x over previous
import jax
import jax.numpy as jnp
from jax.experimental import pallas as pl
from jax.experimental.pallas import tpu as pltpu

_IN_LEN = 64
_C1, _C2, _NOUT = 16, 32, 2
_L1 = _IN_LEN - 2
_P1 = _L1 // 2
_L2 = _P1 - 2
_P2 = _L2 // 2
_PT = 32
_M1W = _PT * _C1
_G = 8
_NG = 4
_W2K = (_G + 2) * _C1
_W2N = 2 * _G * _C1
_FCK = _NG * 4 * _C2
_NPAD = 128
_OUTW = 8


def _pack(w1, b1, w2, b2, fcw, fcb):
    f32 = jnp.float32
    w1 = w1.astype(f32)

    r = jnp.arange(_IN_LEN)[:, None]
    col = jnp.arange(_M1W)[None, :]
    tp = col // _C1
    valid = tp <= _P1 - 1
    w1a = jnp.zeros((_IN_LEN, _M1W), f32)
    w1b = jnp.zeros((_IN_LEN, _M1W), f32)
    for k in range(3):
        vk = jnp.tile(w1[:, 0, k], _PT)[None, :]
        w1a = w1a + jnp.where((r == 2 * tp + k) & valid, vk, 0.0)
        w1b = w1b + jnp.where((r == 2 * tp + 1 + k) & valid, vk, 0.0)
    w1p = jnp.concatenate([w1a, w1b], axis=1)
    b1row = (jnp.tile(b1.astype(f32), _PT) *
             (jnp.arange(_M1W) < _P1 * _C1))[None, :]

    dtp = (jnp.arange(_W2K) // _C1)[:, None]
    cv = jnp.arange(_W2N)[None, :]
    t_loc = 2 * ((cv % 128) // _C2) + (cv >= 128)
    w2p = jnp.zeros((_W2K, _W2N), f32)
    for k in range(3):
        tile_k = jnp.tile(w2[:, :, k].astype(f32).T, (_W2K // _C1, _W2N // _C2))
        w2p = w2p + jnp.where(dtp == t_loc + k, tile_k, 0.0)
    b2row = jnp.tile(b2.astype(f32), 4)[None, :]

    f3 = jnp.transpose(fcw.astype(f32).reshape(_NOUT, _C2, _P2), (2, 1, 0))
    f3 = jnp.pad(f3, ((0, 2), (0, 0), (0, 0))).reshape(_FCK, _NOUT)
    wfc = jnp.pad(f3, ((0, 0), (0, _NPAD - _NOUT)))
    fcbrow = jnp.pad(fcb.astype(f32), (0, _NPAD - _NOUT))[None, :]

    bf16 = jnp.bfloat16
    return (w1p.astype(bf16), b1row.astype(bf16), w2p.astype(bf16),
            b2row.astype(bf16), wfc.astype(bf16), fcbrow)


def _body(x_ref, w1_ref, b1_ref, w2_ref, b2_ref, wfc_ref, fcb_ref, out_ref):
    f32 = jnp.float32
    bf16 = jnp.bfloat16
    x = x_ref[...].astype(bf16)
    y1 = jnp.dot(x, w1_ref[...], preferred_element_type=f32)
    y1 = y1.astype(bf16)
    m1 = jnp.maximum(y1[:, :_M1W], y1[:, _M1W:]) + b1_ref[...]
    m1 = jnp.maximum(m1, 0.0)
    m1 = jnp.concatenate(
        [m1, jnp.zeros((m1.shape[0], 2 * _C1 * _NG), bf16)], axis=1)

    w2 = w2_ref[...]
    parts = []
    for g in range(_NG):
        lo = 2 * _C1 * _G // 2 * g
        a2g = jnp.dot(m1[:, lo:lo + _W2K], w2,
                      preferred_element_type=f32)
        a2g = a2g.astype(bf16)
        m2g = jnp.maximum(a2g[:, :128], a2g[:, 128:]) + b2_ref[...]
        parts.append(jnp.maximum(m2g, 0.0))
    m2 = jnp.concatenate(parts, axis=1)

    y = jnp.dot(m2, wfc_ref[...], preferred_element_type=f32)
    out_ref[...] = (y + fcb_ref[...])[:, :_OUTW]


def kernel(x, w1, b1, w2, b2, fcw, fcb, tb=8192):
    B, L = x.shape
    assert L == _IN_LEN
    w1p, b1row, w2p, b2row, wfc, fcbrow = _pack(w1, b1, w2, b2, fcw, fcb)

    nblk = pl.cdiv(B, tb)
    b_pad = nblk * tb
    x_pad = jnp.pad(x.astype(jnp.float32), ((0, b_pad - B), (0, 0)))
    out = pl.pallas_call(
        _body,
        out_shape=jax.ShapeDtypeStruct((b_pad, _OUTW), jnp.float32),
        grid=(nblk,),
        in_specs=[
            pl.BlockSpec((tb, _IN_LEN), lambda i: (i, 0)),
            pl.BlockSpec((_IN_LEN, 2 * _M1W), lambda i: (0, 0)),
            pl.BlockSpec((1, _M1W), lambda i: (0, 0)),
            pl.BlockSpec((_W2K, _W2N), lambda i: (0, 0)),
            pl.BlockSpec((1, 128), lambda i: (0, 0)),
            pl.BlockSpec((_FCK, _NPAD), lambda i: (0, 0)),
            pl.BlockSpec((1, _NPAD), lambda i: (0, 0)),
        ],
        out_specs=pl.BlockSpec((tb, _OUTW), lambda i: (i, 0)),
        compiler_params=pltpu.CompilerParams(
            dimension_semantics=("parallel",),
            vmem_limit_bytes=64 * 1024 * 1024),
    )(x_pad, w1p, b1row, w2p, b2row, wfc, fcbrow)
    return out[:B, :_NOUT]


# speedup vs baseline: 1.0139x; 1.0008x over previous
import jax
import jax.numpy as jnp
from jax.experimental import pallas as pl
from jax.experimental.pallas import tpu as pltpu

_IN_LEN = 64
_C1, _C2, _NOUT = 16, 32, 2
_L1 = _IN_LEN - 2
_P1 = _L1 // 2
_L2 = _P1 - 2
_P2 = _L2 // 2
_PT = 32
_M1W = _PT * _C1
_G = 8
_NG = 4
_W2K = (_G + 2) * _C1
_W2N = 2 * _G * _C1
_FCK = _NG * 4 * _C2
_NPAD = 128
_OUTW = 8


def _pack(w1, b1, w2, b2, fcw, fcb):
    f32 = jnp.float32
    w1 = w1.astype(f32)

    r = jnp.arange(_IN_LEN)[:, None]
    col = jnp.arange(_M1W)[None, :]
    tp = col // _C1
    valid = tp <= _P1 - 1
    w1a = jnp.zeros((_IN_LEN, _M1W), f32)
    w1b = jnp.zeros((_IN_LEN, _M1W), f32)
    for k in range(3):
        vk = jnp.tile(w1[:, 0, k], _PT)[None, :]
        w1a = w1a + jnp.where((r == 2 * tp + k) & valid, vk, 0.0)
        w1b = w1b + jnp.where((r == 2 * tp + 1 + k) & valid, vk, 0.0)
    w1p = jnp.concatenate([w1a, w1b], axis=1)
    b1row = (jnp.tile(b1.astype(f32), _PT) *
             (jnp.arange(_M1W) < _P1 * _C1))[None, :]

    dtp = (jnp.arange(_W2K) // _C1)[:, None]
    cv = jnp.arange(_W2N)[None, :]
    t_loc = 2 * ((cv % 128) // _C2) + (cv >= 128)
    w2p = jnp.zeros((_W2K, _W2N), f32)
    for k in range(3):
        tile_k = jnp.tile(w2[:, :, k].astype(f32).T, (_W2K // _C1, _W2N // _C2))
        w2p = w2p + jnp.where(dtp == t_loc + k, tile_k, 0.0)
    b2row = jnp.tile(b2.astype(f32), 4)[None, :]

    f3 = jnp.transpose(fcw.astype(f32).reshape(_NOUT, _C2, _P2), (2, 1, 0))
    f3 = jnp.pad(f3, ((0, 2), (0, 0), (0, 0))).reshape(_FCK, _NOUT)
    wfc = jnp.pad(f3, ((0, 0), (0, _NPAD - _NOUT)))
    fcbrow = jnp.pad(fcb.astype(f32), (0, _NPAD - _NOUT))[None, :]

    bf16 = jnp.bfloat16
    return (w1p.astype(bf16), b1row.astype(bf16), w2p.astype(bf16),
            b2row.astype(bf16), wfc.astype(bf16), fcbrow)


def _body(x_ref, w1_ref, b1_ref, w2_ref, b2_ref, wfc_ref, fcb_ref, out_ref):
    f32 = jnp.float32
    bf16 = jnp.bfloat16
    x = x_ref[...].astype(bf16)
    y1 = jnp.dot(x, w1_ref[...], preferred_element_type=f32)
    y1 = y1.astype(bf16)
    m1 = jnp.maximum(y1[:, :_M1W], y1[:, _M1W:]) + b1_ref[...]
    m1 = jnp.maximum(m1, 0.0)

    w2 = w2_ref[...]
    parts = []
    for g in range(_NG):
        lo = 128 * g
        if g < _NG - 1:
            a2g = jnp.dot(m1[:, lo:lo + _W2K], w2,
                          preferred_element_type=f32)
        else:
            a2g = jnp.dot(m1[:, lo:], w2[:_M1W - lo, :],
                          preferred_element_type=f32)
        a2g = a2g.astype(bf16)
        m2g = jnp.maximum(a2g[:, :128], a2g[:, 128:]) + b2_ref[...]
        parts.append(jnp.maximum(m2g, 0.0))
    m2 = jnp.concatenate(parts, axis=1)

    y = jnp.dot(m2, wfc_ref[...], preferred_element_type=f32)
    out_ref[...] = (y + fcb_ref[...])[:, :_OUTW]


def kernel(x, w1, b1, w2, b2, fcw, fcb, tb=8192):
    B, L = x.shape
    assert L == _IN_LEN
    w1p, b1row, w2p, b2row, wfc, fcbrow = _pack(w1, b1, w2, b2, fcw, fcb)

    nblk = pl.cdiv(B, tb)
    b_pad = nblk * tb
    x_pad = jnp.pad(x.astype(jnp.float32), ((0, b_pad - B), (0, 0)))
    out = pl.pallas_call(
        _body,
        out_shape=jax.ShapeDtypeStruct((b_pad, _OUTW), jnp.float32),
        grid=(nblk,),
        in_specs=[
            pl.BlockSpec((tb, _IN_LEN), lambda i: (i, 0)),
            pl.BlockSpec((_IN_LEN, 2 * _M1W), lambda i: (0, 0)),
            pl.BlockSpec((1, _M1W), lambda i: (0, 0)),
            pl.BlockSpec((_W2K, _W2N), lambda i: (0, 0)),
            pl.BlockSpec((1, 128), lambda i: (0, 0)),
            pl.BlockSpec((_FCK, _NPAD), lambda i: (0, 0)),
            pl.BlockSpec((1, _NPAD), lambda i: (0, 0)),
        ],
        out_specs=pl.BlockSpec((tb, _OUTW), lambda i: (i, 0)),
        compiler_params=pltpu.CompilerParams(
            dimension_semantics=("parallel",),
            vmem_limit_bytes=64 * 1024 * 1024),
    )(x_pad, w1p, b1row, w2p, b2row, wfc, fcbrow)
    return out[:B, :_NOUT]


# speedup vs baseline: 1.1698x; 1.1537x over previous
import jax
import jax.numpy as jnp
from jax.experimental import pallas as pl
from jax.experimental.pallas import tpu as pltpu

_IN_LEN = 64
_C1, _C2, _NOUT = 16, 32, 2
_L1 = _IN_LEN - 2
_P1 = _L1 // 2
_L2 = _P1 - 2
_P2 = _L2 // 2
_PT = 32
_M1W = _PT * _C1
_G = 8
_NG = 4
_W2K = (_G + 2) * _C1
_W2N = 2 * _G * _C1
_FCK = _NG * 4 * _C2
_NPAD = 128
_OUTW = 8


def _pack(w1, b1, w2, b2, fcw, fcb):
    f32 = jnp.float32
    w1 = w1.astype(f32)

    r = jnp.arange(_IN_LEN)[:, None]
    col = jnp.arange(_M1W)[None, :]
    tp = col // _C1
    valid = tp <= _P1 - 1
    w1a = jnp.zeros((_IN_LEN, _M1W), f32)
    w1b = jnp.zeros((_IN_LEN, _M1W), f32)
    for k in range(3):
        vk = jnp.tile(w1[:, 0, k], _PT)[None, :]
        w1a = w1a + jnp.where((r == 2 * tp + k) & valid, vk, 0.0)
        w1b = w1b + jnp.where((r == 2 * tp + 1 + k) & valid, vk, 0.0)
    w1p = jnp.concatenate([w1a, w1b], axis=1)
    b1row = (jnp.tile(b1.astype(f32), _PT) *
             (jnp.arange(_M1W) < _P1 * _C1))[None, :]

    dtp = (jnp.arange(_W2K) // _C1)[:, None]
    cv = jnp.arange(_W2N)[None, :]
    t_loc = 2 * ((cv % 128) // _C2) + (cv >= 128)
    w2p = jnp.zeros((_W2K, _W2N), f32)
    for k in range(3):
        tile_k = jnp.tile(w2[:, :, k].astype(f32).T, (_W2K // _C1, _W2N // _C2))
        w2p = w2p + jnp.where(dtp == t_loc + k, tile_k, 0.0)
    b2row = jnp.tile(b2.astype(f32), 4)[None, :]

    f3 = jnp.transpose(fcw.astype(f32).reshape(_NOUT, _C2, _P2), (2, 1, 0))
    f3 = jnp.pad(f3, ((0, 2), (0, 0), (0, 0))).reshape(_FCK, _NOUT)
    wfc = jnp.pad(f3.T, ((0, 8 - _NOUT), (0, 0)))
    fcbrow = jnp.pad(fcb.astype(f32), (0, _NPAD - _NOUT))[None, :]

    bf16 = jnp.bfloat16
    return (w1p.astype(bf16), b1row.astype(bf16), w2p.astype(bf16),
            b2row.astype(bf16), wfc.astype(bf16), fcbrow)


def _body(x_ref, w1_ref, b1_ref, w2_ref, b2_ref, wfc_ref, out_ref):
    f32 = jnp.float32
    bf16 = jnp.bfloat16
    x = x_ref[...].astype(bf16)
    y1 = jnp.dot(x, w1_ref[...], preferred_element_type=f32)
    y1 = y1.astype(bf16)
    m1 = jnp.maximum(y1[:, :_M1W], y1[:, _M1W:]) + b1_ref[...]
    m1 = jnp.maximum(m1, 0.0)

    w2 = w2_ref[...]
    parts = []
    for g in range(_NG):
        lo = 128 * g
        if g < _NG - 1:
            a2g = jnp.dot(m1[:, lo:lo + _W2K], w2,
                          preferred_element_type=f32)
        else:
            a2g = jnp.dot(m1[:, lo:], w2[:_M1W - lo, :],
                          preferred_element_type=f32)
        a2g = a2g.astype(bf16)
        m2g = jnp.maximum(a2g[:, :128], a2g[:, 128:]) + b2_ref[...]
        parts.append(jnp.maximum(m2g, 0.0))
    m2 = jnp.concatenate(parts, axis=1)

    y = jax.lax.dot_general(wfc_ref[...], m2, (((1,), (1,)), ((), ())),
                            preferred_element_type=f32)
    out_ref[...] = y


def kernel(x, w1, b1, w2, b2, fcw, fcb, tb=8192):
    B, L = x.shape
    assert L == _IN_LEN
    w1p, b1row, w2p, b2row, wfc, fcbrow = _pack(w1, b1, w2, b2, fcw, fcb)

    nblk = pl.cdiv(B, tb)
    b_pad = nblk * tb
    x_pad = jnp.pad(x.astype(jnp.float32), ((0, b_pad - B), (0, 0)))
    out = pl.pallas_call(
        _body,
        out_shape=jax.ShapeDtypeStruct((8, b_pad), jnp.float32),
        grid=(nblk,),
        in_specs=[
            pl.BlockSpec((tb, _IN_LEN), lambda i: (i, 0)),
            pl.BlockSpec((_IN_LEN, 2 * _M1W), lambda i: (0, 0)),
            pl.BlockSpec((1, _M1W), lambda i: (0, 0)),
            pl.BlockSpec((_W2K, _W2N), lambda i: (0, 0)),
            pl.BlockSpec((1, 128), lambda i: (0, 0)),
            pl.BlockSpec((8, _FCK), lambda i: (0, 0)),
        ],
        out_specs=pl.BlockSpec((8, tb), lambda i: (0, i)),
        compiler_params=pltpu.CompilerParams(
            dimension_semantics=("parallel",),
            vmem_limit_bytes=64 * 1024 * 1024),
    )(x_pad, w1p, b1row, w2p, b2row, wfc)
    return out[:_NOUT, :B].T + fcb[None, :].astype(jnp.float32)


# speedup vs baseline: 1.1751x; 1.0046x over previous
import jax
import jax.numpy as jnp
from jax.experimental import pallas as pl
from jax.experimental.pallas import tpu as pltpu

_IN_LEN = 64
_C1, _C2, _NOUT = 16, 32, 2
_L1 = _IN_LEN - 2
_P1 = _L1 // 2
_L2 = _P1 - 2
_P2 = _L2 // 2
_PT = 32
_M1W = _PT * _C1
_G = 8
_NG = 4
_W2K = (_G + 2) * _C1
_W2N = 2 * _G * _C1
_FCK = _NG * 4 * _C2


def _pack(w1, b1, w2, b2, fcw, fcb):
    f32 = jnp.float32
    w1 = w1.astype(f32)

    r = jnp.arange(_IN_LEN)[:, None]
    col = jnp.arange(_M1W)[None, :]
    tp = col // _C1
    valid = tp <= _P1 - 1
    w1a = jnp.zeros((_IN_LEN, _M1W), f32)
    w1b = jnp.zeros((_IN_LEN, _M1W), f32)
    for k in range(3):
        vk = jnp.tile(w1[:, 0, k], _PT)[None, :]
        w1a = w1a + jnp.where((r == 2 * tp + k) & valid, vk, 0.0)
        w1b = w1b + jnp.where((r == 2 * tp + 1 + k) & valid, vk, 0.0)
    w1p = jnp.concatenate([w1a, w1b], axis=1)
    b1row = (jnp.tile(b1.astype(f32), _PT) *
             (jnp.arange(_M1W) < _P1 * _C1))[None, :]

    dtp = (jnp.arange(_W2K) // _C1)[:, None]
    cv = jnp.arange(_W2N)[None, :]
    t_loc = 2 * ((cv % 128) // _C2) + (cv >= 128)
    w2p = jnp.zeros((_W2K, _W2N), f32)
    for k in range(3):
        tile_k = jnp.tile(w2[:, :, k].astype(f32).T, (_W2K // _C1, _W2N // _C2))
        w2p = w2p + jnp.where(dtp == t_loc + k, tile_k, 0.0)
    b2row = jnp.tile(b2.astype(f32), 4)[None, :]

    f3 = jnp.transpose(fcw.astype(f32).reshape(_NOUT, _C2, _P2), (2, 1, 0))
    f3 = jnp.pad(f3, ((0, 2), (0, 0), (0, 0))).reshape(_FCK, _NOUT)
    wfc = jnp.pad(f3.T, ((0, 8 - _NOUT), (0, 0)))

    bf16 = jnp.bfloat16
    return (w1p.astype(bf16), b1row, w2p.astype(bf16), b2row,
            wfc.astype(bf16))


def _body(x_ref, w1_ref, b1_ref, w2_ref, b2_ref, wfc_ref, out_ref):
    f32 = jnp.float32
    bf16 = jnp.bfloat16
    x = x_ref[...].astype(bf16)
    y1 = jnp.dot(x, w1_ref[...], preferred_element_type=f32)
    m1 = jnp.maximum(y1[:, :_M1W], y1[:, _M1W:]) + b1_ref[...]
    m1 = jnp.maximum(m1, 0.0).astype(bf16)

    w2 = w2_ref[...]
    parts = []
    for g in range(_NG):
        lo = 128 * g
        if g < _NG - 1:
            a2g = jnp.dot(m1[:, lo:lo + _W2K], w2,
                          preferred_element_type=f32)
        else:
            a2g = jnp.dot(m1[:, lo:], w2[:_M1W - lo, :],
                          preferred_element_type=f32)
        m2g = jnp.maximum(a2g[:, :128], a2g[:, 128:]) + b2_ref[...]
        parts.append(jnp.maximum(m2g, 0.0).astype(bf16))
    m2 = jnp.concatenate(parts, axis=1)

    y = jax.lax.dot_general(wfc_ref[...], m2, (((1,), (1,)), ((), ())),
                            preferred_element_type=f32)
    out_ref[...] = y


def kernel(x, w1, b1, w2, b2, fcw, fcb, tb=8192):
    B, L = x.shape
    assert L == _IN_LEN
    w1p, b1row, w2p, b2row, wfc = _pack(w1, b1, w2, b2, fcw, fcb)

    nblk = pl.cdiv(B, tb)
    b_pad = nblk * tb
    x_pad = jnp.pad(x.astype(jnp.float32), ((0, b_pad - B), (0, 0)))
    out = pl.pallas_call(
        _body,
        out_shape=jax.ShapeDtypeStruct((8, b_pad), jnp.float32),
        grid=(nblk,),
        in_specs=[
            pl.BlockSpec((tb, _IN_LEN), lambda i: (i, 0)),
            pl.BlockSpec((_IN_LEN, 2 * _M1W), lambda i: (0, 0)),
            pl.BlockSpec((1, _M1W), lambda i: (0, 0)),
            pl.BlockSpec((_W2K, _W2N), lambda i: (0, 0)),
            pl.BlockSpec((1, 128), lambda i: (0, 0)),
            pl.BlockSpec((8, _FCK), lambda i: (0, 0)),
        ],
        out_specs=pl.BlockSpec((8, tb), lambda i: (0, i)),
        compiler_params=pltpu.CompilerParams(
            dimension_semantics=("parallel",),
            vmem_limit_bytes=64 * 1024 * 1024),
    )(x_pad, w1p, b1row, w2p, b2row, wfc)
    return out[:_NOUT, :B].T + fcb[None, :].astype(jnp.float32)
